# baseline (device time: 65029 ns/iter reference)
import jax
import jax.numpy as jnp
from jax import lax
from jax.experimental import pallas as pl
from jax.experimental.pallas import tpu as pltpu

M_PER = 4096
N = 1024
HALF = M_PER // 2
K = 32
CH = HALF // K


def kernel(x):
    def body(x_ref, out_ref, buf, send_a, recv_a, send_b, recv_b, local_sem):
        my_x = lax.axis_index("x")
        my_y = lax.axis_index("y")
        other_x = 1 - my_x
        other_y = 1 - my_y

        barrier_sem = pltpu.get_barrier_semaphore()
        pl.semaphore_signal(barrier_sem, inc=1, device_id=(other_x, my_y),
                            device_id_type=pl.DeviceIdType.MESH)
        pl.semaphore_signal(barrier_sem, inc=1, device_id=(my_x, other_y),
                            device_id_type=pl.DeviceIdType.MESH)
        pl.semaphore_wait(barrier_sem, 2)

        my_half_rows = my_x * M_PER + my_y * HALF
        rdma_a = []
        for i in range(K):
            r0 = my_y * HALF + i * CH
            buf[pl.ds(r0, CH), :] = x_ref[pl.ds(r0, CH), :].astype(jnp.bfloat16)
            r = pltpu.make_async_remote_copy(
                src_ref=buf.at[pl.ds(r0, CH), :],
                dst_ref=out_ref.at[pl.ds(my_half_rows + i * CH, CH), :],
                send_sem=send_a.at[i],
                recv_sem=recv_a.at[i],
                device_id=(other_x, my_y),
                device_id_type=pl.DeviceIdType.MESH,
            )
            r.start()
            rdma_a.append(r)

        oh = other_y * HALF
        buf[pl.ds(oh, HALF), :] = x_ref[pl.ds(oh, HALF), :].astype(jnp.bfloat16)
        local_copy = pltpu.make_async_copy(
            buf, out_ref.at[pl.ds(my_x * M_PER, M_PER), :], local_sem
        )
        local_copy.start()

        recvd_rows = other_x * M_PER + my_y * HALF
        rdma_b = []
        for i in range(K):
            rdma_a[i].wait_recv()
            rows = recvd_rows + i * CH
            r = pltpu.make_async_remote_copy(
                src_ref=out_ref.at[pl.ds(rows, CH), :],
                dst_ref=out_ref.at[pl.ds(rows, CH), :],
                send_sem=send_b.at[i],
                recv_sem=recv_b.at[i],
                device_id=(my_x, other_y),
                device_id_type=pl.DeviceIdType.MESH,
            )
            r.start()
            rdma_b.append(r)

        for i in range(K):
            rdma_b[i].wait_recv()
        for i in range(K):
            rdma_a[i].wait_send()
            rdma_b[i].wait_send()
        local_copy.wait()

    return pl.pallas_call(
        body,
        out_shape=jax.ShapeDtypeStruct((2 * M_PER, N), jnp.bfloat16),
        in_specs=[pl.BlockSpec(memory_space=pltpu.VMEM)],
        out_specs=pl.BlockSpec(memory_space=pltpu.MemorySpace.HBM),
        scratch_shapes=[
            pltpu.VMEM((M_PER, N), jnp.bfloat16),
            pltpu.SemaphoreType.DMA((K,)),
            pltpu.SemaphoreType.DMA((K,)),
            pltpu.SemaphoreType.DMA((K,)),
            pltpu.SemaphoreType.DMA((K,)),
            pltpu.SemaphoreType.DMA,
        ],
        compiler_params=pltpu.CompilerParams(collective_id=0),
    )(x)


# device time: 61598 ns/iter; 1.0557x vs baseline; 1.0557x over previous
import jax
import jax.numpy as jnp
from jax import lax
from jax.experimental import pallas as pl
from jax.experimental.pallas import tpu as pltpu

M_PER = 4096
N = 1024
HALF = M_PER // 2
K = 16
CH = HALF // K
SUB = 4
SUBR = HALF // SUB
CPS = SUBR // CH


def kernel(x):
    def body(x_ref, out_ref, xvm, buf, send_a, recv_a, send_b, recv_b,
             in_sems, local_sem):
        my_x = lax.axis_index("x")
        my_y = lax.axis_index("y")
        other_x = 1 - my_x
        other_y = 1 - my_y

        in_dmas = []
        for j in range(2 * SUB):
            half = my_y * HALF if j < SUB else other_y * HALF
            r0 = half + (j % SUB) * SUBR
            c = pltpu.make_async_copy(
                x_ref.at[pl.ds(r0, SUBR), :], xvm.at[pl.ds(r0, SUBR), :],
                in_sems.at[j],
            )
            c.start()
            in_dmas.append(c)

        barrier_sem = pltpu.get_barrier_semaphore()
        pl.semaphore_signal(barrier_sem, inc=1, device_id=(other_x, my_y),
                            device_id_type=pl.DeviceIdType.MESH)
        pl.semaphore_signal(barrier_sem, inc=1, device_id=(my_x, other_y),
                            device_id_type=pl.DeviceIdType.MESH)
        pl.semaphore_wait(barrier_sem, 2)

        my_half_rows = my_x * M_PER + my_y * HALF
        rdma_a = []
        for i in range(K):
            if i % CPS == 0:
                in_dmas[i // CPS].wait()
            r0 = my_y * HALF + i * CH
            buf[pl.ds(r0, CH), :] = xvm[pl.ds(r0, CH), :].astype(jnp.bfloat16)
            r = pltpu.make_async_remote_copy(
                src_ref=buf.at[pl.ds(r0, CH), :],
                dst_ref=out_ref.at[pl.ds(my_half_rows + i * CH, CH), :],
                send_sem=send_a.at[i],
                recv_sem=recv_a.at[i],
                device_id=(other_x, my_y),
                device_id_type=pl.DeviceIdType.MESH,
            )
            r.start()
            rdma_a.append(r)

        for j in range(SUB):
            in_dmas[SUB + j].wait()
            r0 = other_y * HALF + j * SUBR
            buf[pl.ds(r0, SUBR), :] = (
                xvm[pl.ds(r0, SUBR), :].astype(jnp.bfloat16)
            )
        local_copy = pltpu.make_async_copy(
            buf, out_ref.at[pl.ds(my_x * M_PER, M_PER), :], local_sem
        )
        local_copy.start()

        recvd_rows = other_x * M_PER + my_y * HALF
        rdma_b = []
        for i in range(K):
            rdma_a[i].wait_recv()
            rows = recvd_rows + i * CH
            r = pltpu.make_async_remote_copy(
                src_ref=out_ref.at[pl.ds(rows, CH), :],
                dst_ref=out_ref.at[pl.ds(rows, CH), :],
                send_sem=send_b.at[i],
                recv_sem=recv_b.at[i],
                device_id=(my_x, other_y),
                device_id_type=pl.DeviceIdType.MESH,
            )
            r.start()
            rdma_b.append(r)

        for i in range(K):
            rdma_b[i].wait_recv()
        for i in range(K):
            rdma_a[i].wait_send()
            rdma_b[i].wait_send()
        local_copy.wait()

    return pl.pallas_call(
        body,
        out_shape=jax.ShapeDtypeStruct((2 * M_PER, N), jnp.bfloat16),
        in_specs=[pl.BlockSpec(memory_space=pl.ANY)],
        out_specs=pl.BlockSpec(memory_space=pl.ANY),
        scratch_shapes=[
            pltpu.VMEM((M_PER, N), jnp.float32),
            pltpu.VMEM((M_PER, N), jnp.bfloat16),
            pltpu.SemaphoreType.DMA((K,)),
            pltpu.SemaphoreType.DMA((K,)),
            pltpu.SemaphoreType.DMA((K,)),
            pltpu.SemaphoreType.DMA((K,)),
            pltpu.SemaphoreType.DMA((2 * SUB,)),
            pltpu.SemaphoreType.DMA,
        ],
        compiler_params=pltpu.CompilerParams(collective_id=0),
    )(x)


# device time: 61566 ns/iter; 1.0562x vs baseline; 1.0005x over previous
import jax
import jax.numpy as jnp
from jax import lax
from jax.experimental import pallas as pl
from jax.experimental.pallas import tpu as pltpu

M_PER = 4096
N = 1024
HALF = M_PER // 2
K = 16
CH = HALF // K
SUB = 4
SUBR = HALF // SUB
CPS = SUBR // CH


def kernel(x):
    def body(x_ref, out_ref, xvm, buf, send_a, recv_a, send_b, recv_b,
             in_sems, local_sem):
        my_x = lax.axis_index("x")
        my_y = lax.axis_index("y")
        other_x = 1 - my_x
        other_y = 1 - my_y

        in_dmas = []
        for j in range(2 * SUB):
            half = my_y * HALF if j < SUB else other_y * HALF
            r0 = half + (j % SUB) * SUBR
            c = pltpu.make_async_copy(
                x_ref.at[pl.ds(r0, SUBR), :], xvm.at[pl.ds(r0, SUBR), :],
                in_sems.at[j],
            )
            c.start()
            in_dmas.append(c)

        barrier_sem = pltpu.get_barrier_semaphore()
        pl.semaphore_signal(barrier_sem, inc=1, device_id=(other_x, my_y),
                            device_id_type=pl.DeviceIdType.MESH)
        pl.semaphore_signal(barrier_sem, inc=1, device_id=(my_x, other_y),
                            device_id_type=pl.DeviceIdType.MESH)
        pl.semaphore_wait(barrier_sem, 2)

        my_half_rows = my_x * M_PER + my_y * HALF
        rdma_a = []
        for i in range(K):
            if i % CPS == 0:
                in_dmas[i // CPS].wait()
            r0 = my_y * HALF + i * CH
            buf[pl.ds(r0, CH), :] = xvm[pl.ds(r0, CH), :].astype(jnp.bfloat16)
            r = pltpu.make_async_remote_copy(
                src_ref=buf.at[pl.ds(r0, CH), :],
                dst_ref=out_ref.at[pl.ds(my_half_rows + i * CH, CH), :],
                send_sem=send_a.at[i],
                recv_sem=recv_a.at[i],
                device_id=(other_x, my_y),
                device_id_type=pl.DeviceIdType.MESH,
            )
            r.start()
            rdma_a.append(r)

        for j in range(SUB):
            in_dmas[SUB + j].wait()
            r0 = other_y * HALF + j * SUBR
            buf[pl.ds(r0, SUBR), :] = (
                xvm[pl.ds(r0, SUBR), :].astype(jnp.bfloat16)
            )
        local_copy = pltpu.make_async_copy(
            buf, out_ref.at[pl.ds(my_x * M_PER, M_PER), :], local_sem
        )
        local_copy.start()

        recvd_rows = other_x * M_PER + my_y * HALF
        rdma_b = []
        for i in range(K):
            rdma_a[i].wait_recv()
            rows = recvd_rows + i * CH
            r = pltpu.make_async_remote_copy(
                src_ref=out_ref.at[pl.ds(rows, CH), :],
                dst_ref=out_ref.at[pl.ds(rows, CH), :],
                send_sem=send_b.at[i],
                recv_sem=recv_b.at[i],
                device_id=(my_x, other_y),
                device_id_type=pl.DeviceIdType.MESH,
            )
            r.start()
            rdma_b.append(r)

        for i in range(K):
            rdma_b[i].wait_recv()
        for i in range(K):
            rdma_a[i].wait_send()
            rdma_b[i].wait_send()
        local_copy.wait()

    out = pl.pallas_call(
        body,
        out_shape=jax.ShapeDtypeStruct((2 * M_PER, N), jnp.bfloat16),
        in_specs=[pl.BlockSpec(memory_space=pl.ANY)],
        out_specs=pl.BlockSpec(memory_space=pl.ANY),
        scratch_shapes=[
            pltpu.VMEM((M_PER, N), jnp.float32),
            pltpu.VMEM((M_PER, N), jnp.bfloat16),
            pltpu.SemaphoreType.DMA((K,)),
            pltpu.SemaphoreType.DMA((K,)),
            pltpu.SemaphoreType.DMA((K,)),
            pltpu.SemaphoreType.DMA((K,)),
            pltpu.SemaphoreType.DMA((2 * SUB,)),
            pltpu.SemaphoreType.DMA,
        ],
        compiler_params=pltpu.CompilerParams(collective_id=0),
    )(x)

    return pltpu.with_memory_space_constraint(out, pltpu.MemorySpace.HBM)
